# Initial kernel scaffold; baseline (speedup 1.0000x reference)
import functools

import jax
import jax.numpy as jnp
from jax import lax
from jax.experimental import pallas as pl
from jax.experimental.pallas import tpu as pltpu
from jax.experimental.pallas import tpu_sc as plsc

D = 64
NC = 2
NS = 16
NW = NC * NS
CHUNK = 128


def _emb_body(table_hbm, idx_hbm, out_hbm, idx_v, rows_v, sem, *, b_per_w,
              n_chunks):
    wid = lax.axis_index("s") * NC + lax.axis_index("c")
    base = wid * b_per_w
    pltpu.sync_copy(idx_hbm.at[pl.ds(base, b_per_w)], idx_v)

    def body(g, carry):
        start = g * CHUNK
        pltpu.async_copy(
            table_hbm.at[idx_v.at[pl.ds(start, CHUNK)]], rows_v, sem
        ).wait()
        pltpu.sync_copy(rows_v, out_hbm.at[pl.ds(base + start, CHUNK)])
        return carry

    lax.fori_loop(0, n_chunks, body, 0)


def kernel(z, table):
    B0, B1 = z.shape
    B = B0 * B1
    idx = z.reshape(B).astype(jnp.int32)
    b_per_w = B // NW
    n_chunks = b_per_w // CHUNK

    mesh = plsc.VectorSubcoreMesh(core_axis_name="c", subcore_axis_name="s")
    emb = functools.partial(
        pl.kernel,
        mesh=mesh,
        out_type=jax.ShapeDtypeStruct((B, D), jnp.float32),
        scratch_types=[
            pltpu.VMEM((b_per_w,), jnp.int32),
            pltpu.VMEM((CHUNK, D), jnp.float32),
            pltpu.SemaphoreType.DMA,
        ],
    )(functools.partial(_emb_body, b_per_w=b_per_w, n_chunks=n_chunks))

    out = emb(table, idx)
    return out.reshape(B0, B1, D)


# speedup vs baseline: 1.6861x; 1.6861x over previous
import functools

import jax
import jax.numpy as jnp
from jax import lax
from jax.experimental import pallas as pl
from jax.experimental.pallas import tpu as pltpu
from jax.experimental.pallas import tpu_sc as plsc

D = 64
NC = 2
NS = 16
NW = NC * NS
CHUNK = 128


def _emb_body(table_hbm, idx_hbm, out_hbm, idx_v, rows_v, sem, *, b_per_w,
              n_chunks):
    wid = lax.axis_index("s") * NC + lax.axis_index("c")
    base = wid * b_per_w
    pltpu.sync_copy(idx_hbm.at[pl.ds(base, b_per_w)], idx_v)

    def body(g, carry):
        start = g * CHUNK
        pltpu.async_copy(
            table_hbm.at[idx_v.at[pl.ds(start, CHUNK)]], rows_v, sem
        ).wait()
        pltpu.sync_copy(rows_v, out_hbm.at[pl.ds(base + start, CHUNK)])
        return carry

    lax.fori_loop(0, n_chunks, body, 0)


def kernel(z, table):
    B0, B1 = z.shape
    B = B0 * B1
    idx = z.reshape(B).astype(jnp.int32)
    b_per_w = B // NW
    n_chunks = b_per_w // CHUNK

    mesh = plsc.VectorSubcoreMesh(core_axis_name="c", subcore_axis_name="s")
    emb = functools.partial(
        pl.kernel,
        mesh=mesh,
        out_type=jax.ShapeDtypeStruct((B, D), jnp.float32),
        scratch_types=[
            pltpu.VMEM((b_per_w,), jnp.int32),
            pltpu.VMEM((CHUNK, D), jnp.float32),
            pltpu.SemaphoreType.DMA,
        ],
        compiler_params=pltpu.CompilerParams(use_tc_tiling_on_sc=False),
    )(functools.partial(_emb_body, b_per_w=b_per_w, n_chunks=n_chunks))

    out = emb(table, idx)
    return out.reshape(B0, B1, D)


# speedup vs baseline: 1.8715x; 1.1099x over previous
import functools

import jax
import jax.numpy as jnp
from jax import lax
from jax.experimental import pallas as pl
from jax.experimental.pallas import tpu as pltpu
from jax.experimental.pallas import tpu_sc as plsc

D = 64
NC = 2
NS = 16
NW = NC * NS
CHUNK = 128
K = 4


def _emb_body(table_hbm, idx_hbm, out_hbm, idx_v, rows_v, gsem, ssem, *,
              b_per_w, n_groups):
    wid = lax.axis_index("s") * NC + lax.axis_index("c")
    base = wid * b_per_w
    pltpu.sync_copy(idx_hbm.at[pl.ds(base, b_per_w)], idx_v)

    def gather_desc(set_off, chunk0, j):
        return pltpu.make_async_copy(
            table_hbm.at[idx_v.at[pl.ds((chunk0 + j) * CHUNK, CHUNK)]],
            rows_v.at[set_off + j],
            gsem,
        )

    def store_desc(set_off, chunk0, j):
        return pltpu.make_async_copy(
            rows_v.at[set_off + j],
            out_hbm.at[pl.ds(base + (chunk0 + j) * CHUNK, CHUNK)],
            ssem,
        )

    for j in range(K):
        gather_desc(0, 0, j).start()

    def body(t, carry):
        prev_off = ((t - 1) % 2) * K
        cur_off = (t % 2) * K
        for j in range(K):
            gather_desc(prev_off, (t - 1) * K, j).wait()
        for j in range(K):
            store_desc(prev_off, (t - 1) * K, j).start()
        @pl.when(t >= 2)
        def _():
            for j in range(K):
                store_desc(cur_off, (t - 2) * K, j).wait()

        for j in range(K):
            gather_desc(cur_off, t * K, j).start()
        return carry

    lax.fori_loop(1, n_groups, body, 0)

    last = n_groups - 1
    last_off = (last % 2) * K
    for j in range(K):
        gather_desc(last_off, last * K, j).wait()
    for j in range(K):
        store_desc(last_off, last * K, j).start()
    prev_off = ((last - 1) % 2) * K
    for j in range(K):
        store_desc(prev_off, (last - 1) * K, j).wait()
    for j in range(K):
        store_desc(last_off, last * K, j).wait()


def kernel(z, table):
    B0, B1 = z.shape
    B = B0 * B1
    idx = z.reshape(B).astype(jnp.int32)
    b_per_w = B // NW
    n_groups = b_per_w // (CHUNK * K)

    mesh = plsc.VectorSubcoreMesh(core_axis_name="c", subcore_axis_name="s")
    emb = functools.partial(
        pl.kernel,
        mesh=mesh,
        out_type=jax.ShapeDtypeStruct((B, D), jnp.float32),
        scratch_types=[
            pltpu.VMEM((b_per_w,), jnp.int32),
            pltpu.VMEM((2 * K, CHUNK, D), jnp.float32),
            pltpu.SemaphoreType.DMA,
            pltpu.SemaphoreType.DMA,
        ],
        compiler_params=pltpu.CompilerParams(use_tc_tiling_on_sc=False),
    )(functools.partial(_emb_body, b_per_w=b_per_w, n_groups=n_groups))

    out = emb(table, idx)
    return out.reshape(B0, B1, D)


# speedup vs baseline: 1.8742x; 1.0015x over previous
import functools

import jax
import jax.numpy as jnp
from jax import lax
from jax.experimental import pallas as pl
from jax.experimental.pallas import tpu as pltpu
from jax.experimental.pallas import tpu_sc as plsc

D = 64
NC = 2
NS = 16
NW = NC * NS
CHUNK = 128
K = 5


def _emb_body(table_hbm, idx_hbm, out_hbm, idx_v, rows_v, gsem, ssem, *,
              b_per_w, n_groups):
    wid = lax.axis_index("s") * NC + lax.axis_index("c")
    base = wid * b_per_w
    pltpu.sync_copy(idx_hbm.at[pl.ds(base, b_per_w)], idx_v)

    def gather_desc(set_off, chunk0, j):
        return pltpu.make_async_copy(
            table_hbm.at[idx_v.at[pl.ds((chunk0 + j) * CHUNK, CHUNK)]],
            rows_v.at[set_off + j],
            gsem,
        )

    def store_desc(set_off, chunk0, j):
        return pltpu.make_async_copy(
            rows_v.at[set_off + j],
            out_hbm.at[pl.ds(base + (chunk0 + j) * CHUNK, CHUNK)],
            ssem,
        )

    for j in range(K):
        gather_desc(0, 0, j).start()

    def body(t, carry):
        prev_off = ((t - 1) % 2) * K
        cur_off = (t % 2) * K
        for j in range(K):
            gather_desc(prev_off, (t - 1) * K, j).wait()
        for j in range(K):
            store_desc(prev_off, (t - 1) * K, j).start()
        @pl.when(t >= 2)
        def _():
            for j in range(K):
                store_desc(cur_off, (t - 2) * K, j).wait()

        for j in range(K):
            gather_desc(cur_off, t * K, j).start()
        return carry

    lax.fori_loop(1, n_groups, body, 0)

    last = n_groups - 1
    last_off = (last % 2) * K
    for j in range(K):
        gather_desc(last_off, last * K, j).wait()
    for j in range(K):
        store_desc(last_off, last * K, j).start()
    prev_off = ((last - 1) % 2) * K
    for j in range(K):
        store_desc(prev_off, (last - 1) * K, j).wait()
    for j in range(K):
        store_desc(last_off, last * K, j).wait()


def kernel(z, table):
    B0, B1 = z.shape
    B = B0 * B1
    idx = z.reshape(B).astype(jnp.int32)
    b_per_w = B // NW
    n_groups = b_per_w // (CHUNK * K)

    mesh = plsc.VectorSubcoreMesh(core_axis_name="c", subcore_axis_name="s")
    emb = functools.partial(
        pl.kernel,
        mesh=mesh,
        out_type=jax.ShapeDtypeStruct((B, D), jnp.float32),
        scratch_types=[
            pltpu.VMEM((b_per_w,), jnp.int32),
            pltpu.VMEM((2 * K, CHUNK, D), jnp.float32),
            pltpu.SemaphoreType.DMA,
            pltpu.SemaphoreType.DMA,
        ],
        compiler_params=pltpu.CompilerParams(use_tc_tiling_on_sc=False),
    )(functools.partial(_emb_body, b_per_w=b_per_w, n_groups=n_groups))

    out = emb(table, idx)
    return out.reshape(B0, B1, D)
